# Initial kernel scaffold; baseline (speedup 1.0000x reference)
import jax
import jax.numpy as jnp
from jax.experimental import pallas as pl


def kernel(input, pe):
    raise NotImplementedError("write your pallas kernel here")


# speedup vs baseline: 1.5762x; 1.5762x over previous
import functools

import jax
import jax.numpy as jnp
from jax import lax
from jax.experimental import pallas as pl
from jax.experimental.pallas import tpu as pltpu
from jax.experimental.pallas import tpu_sc as plsc

BSZ = 4
SEQ = 4096
D_MODEL = 1024
NC = 2
NS = 16
NW = NC * NS
ROWS_PER_W = SEQ // NW
CHUNK = 64
NCHUNK = ROWS_PER_W // CHUNK


def _pe_broadcast_body(pe_hbm, out_hbm, buf, sem):
    wid = lax.axis_index("s") * NC + lax.axis_index("c")
    base = wid * ROWS_PER_W
    for i in range(NCHUNK):
        row = base + i * CHUNK
        pltpu.async_copy(pe_hbm.at[pl.ds(row, CHUNK)], buf, sem).wait()
        for b in range(BSZ):
            pltpu.sync_copy(buf, out_hbm.at[b, pl.ds(row, CHUNK)])


@jax.jit
def _pe_broadcast(pe):
    mesh = plsc.VectorSubcoreMesh(core_axis_name="c", subcore_axis_name="s")
    f = pl.kernel(
        _pe_broadcast_body,
        mesh=mesh,
        out_type=jax.ShapeDtypeStruct((BSZ, SEQ, D_MODEL), jnp.float32),
        scratch_types=[
            pltpu.VMEM((CHUNK, D_MODEL), jnp.float32),
            pltpu.SemaphoreType.DMA,
        ],
    )
    return f(pe)


def kernel(input, pe):
    del input
    return _pe_broadcast(pe)


# speedup vs baseline: 1.6245x; 1.0306x over previous
import functools

import jax
import jax.numpy as jnp
from jax import lax
from jax.experimental import pallas as pl
from jax.experimental.pallas import tpu as pltpu
from jax.experimental.pallas import tpu_sc as plsc

BSZ = 4
SEQ = 4096
D_MODEL = 1024
NC = 2
NS = 16
NW = NC * NS
ROWS_PER_W = SEQ // NW
CHUNK = 64
NCHUNK = ROWS_PER_W // CHUNK


def _pe_broadcast_body(pe_hbm, out_hbm, buf0, buf1, sem_r0, sem_r1, sem_w):
    wid = lax.axis_index("s") * NC + lax.axis_index("c")
    base = wid * ROWS_PER_W
    r0 = pltpu.async_copy(pe_hbm.at[pl.ds(base, CHUNK)], buf0, sem_r0)
    r1 = pltpu.async_copy(pe_hbm.at[pl.ds(base + CHUNK, CHUNK)], buf1, sem_r1)
    writes = []
    r0.wait()
    for b in range(BSZ):
        writes.append(pltpu.async_copy(buf0, out_hbm.at[b, pl.ds(base, CHUNK)], sem_w))
    r1.wait()
    for b in range(BSZ):
        writes.append(pltpu.async_copy(buf1, out_hbm.at[b, pl.ds(base + CHUNK, CHUNK)], sem_w))
    for w in writes:
        w.wait()


@jax.jit
def _pe_broadcast(pe):
    mesh = plsc.VectorSubcoreMesh(core_axis_name="c", subcore_axis_name="s")
    f = pl.kernel(
        _pe_broadcast_body,
        mesh=mesh,
        out_type=jax.ShapeDtypeStruct((BSZ, SEQ, D_MODEL), jnp.float32),
        scratch_types=[
            pltpu.VMEM((CHUNK, D_MODEL), jnp.float32),
            pltpu.VMEM((CHUNK, D_MODEL), jnp.float32),
            pltpu.SemaphoreType.DMA,
            pltpu.SemaphoreType.DMA,
            pltpu.SemaphoreType.DMA,
        ],
    )
    return f(pe)


def kernel(input, pe):
    del input
    return _pe_broadcast(pe)
